# Initial kernel scaffold; baseline (speedup 1.0000x reference)
import jax
import jax.numpy as jnp
from jax.experimental import pallas as pl


def kernel(predicted_locs, predicted_scores, boxes, labels, priors_cxcy):
    raise NotImplementedError("write your pallas kernel here")


# speedup vs baseline: 23.9862x; 23.9862x over previous
import jax
import jax.numpy as jnp
from jax.experimental import pallas as pl

_THRESHOLD = 0.5
_NEG_POS_RATIO = 3
_ALPHA = 1.0


def _phase1_body(boxes_ref, labels_ref, priors_ref, locs_ref, scores_ref,
                 conf_neg_ref, stats_ref):
    nobj = boxes_ref.shape[1]
    num_classes = scores_ref.shape[1]
    num_priors = priors_ref.shape[1]

    boxes = boxes_ref[0]
    labels = labels_ref[0]
    pr = priors_ref[...]
    pcx, pcy, pw, ph = pr[0:1], pr[1:2], pr[2:3], pr[3:4]
    px1 = pcx - pw * 0.5
    py1 = pcy - ph * 0.5
    px2 = pcx + pw * 0.5
    py2 = pcy + ph * 0.5

    bx1 = boxes[:, 0:1]
    by1 = boxes[:, 1:2]
    bx2 = boxes[:, 2:3]
    by2 = boxes[:, 3:4]

    iw = jnp.maximum(jnp.minimum(bx2, px2) - jnp.maximum(bx1, px1), 0.0)
    ih = jnp.maximum(jnp.minimum(by2, py2) - jnp.maximum(by1, py1), 0.0)
    inter = iw * ih
    area_a = (bx2 - bx1) * (by2 - by1)
    area_b = (px2 - px1) * (py2 - py1)
    ov = inter / (area_a + area_b - inter)

    ji = jax.lax.broadcasted_iota(jnp.int32, (nobj, num_priors), 0)
    li = jax.lax.broadcasted_iota(jnp.int32, (nobj, num_priors), 1)
    big = jnp.int32(2**30)

    colmax = jnp.max(ov, axis=0, keepdims=True)
    besti = jnp.min(jnp.where(ov == colmax, ji, big), axis=0, keepdims=True)
    rowmax = jnp.max(ov, axis=1, keepdims=True)
    pox = jnp.min(jnp.where(ov == rowmax, li, big), axis=1, keepdims=True)
    eq = li == pox
    jstar = jnp.max(jnp.where(eq, ji, -1), axis=0, keepdims=True)
    forced = jstar >= 0
    obj = jnp.where(forced, jstar, besti)
    ovl = jnp.where(forced, 1.0, colmax)

    sel_i = (obj == ji).astype(jnp.int32)
    sel_f = sel_i.astype(jnp.float32)
    lab = jnp.sum(sel_i * labels, axis=0, keepdims=True)
    lab = jnp.where(ovl < _THRESHOLD, 0, lab)
    non_bck = lab > 0
    maskf = non_bck.astype(jnp.float32)

    bcx = (bx1 + bx2) * 0.5
    bcy = (by1 + by2) * 0.5
    bw = bx2 - bx1
    bh = by2 - by1
    scx = jnp.sum(sel_f * bcx, axis=0, keepdims=True)
    scy = jnp.sum(sel_f * bcy, axis=0, keepdims=True)
    sbw = jnp.sum(sel_f * bw, axis=0, keepdims=True)
    sbh = jnp.sum(sel_f * bh, axis=0, keepdims=True)

    g_cx = 10.0 * (scx - pcx) / pw
    g_cy = 10.0 * (scy - pcy) / ph
    g_w = 5.0 * jnp.log(sbw / pw)
    g_h = 5.0 * jnp.log(sbh / ph)

    locs = locs_ref[0]
    l1 = (jnp.abs(locs[0:1] - g_cx) + jnp.abs(locs[1:2] - g_cy)
          + jnp.abs(locs[2:3] - g_w) + jnp.abs(locs[3:4] - g_h)) * maskf
    loc_abs = jnp.sum(l1)
    npos = jnp.sum(maskf)

    s = scores_ref[0]
    m = jnp.max(s, axis=0, keepdims=True)
    sse = jnp.sum(jnp.exp(s - m), axis=0, keepdims=True)
    lse = jnp.log(sse) + m
    ci = jax.lax.broadcasted_iota(jnp.int32, (num_classes, num_priors), 0)
    s_true = jnp.sum(jnp.where(ci == lab, s, 0.0), axis=0, keepdims=True)
    ce = lse - s_true

    cep = jnp.sum(jnp.where(non_bck, ce, 0.0))
    cneg = jnp.where(non_bck, 0.0, jnp.maximum(ce, 0.0))
    conf_neg_ref[0] = cneg

    lanes = jax.lax.broadcasted_iota(jnp.int32, (1, 128), 1)
    stats = jnp.where(lanes == 0, npos,
                      jnp.where(lanes == 1, loc_abs,
                                jnp.where(lanes == 2, cep, 0.0)))
    stats_ref[0] = stats


def _phase2_body(conf_neg_ref, stats_ref, out_ref):
    batch = conf_neg_ref.shape[0]
    num_priors = conf_neg_ref.shape[2]
    x = conf_neg_ref[...].reshape(batch, num_priors)
    st = stats_ref[...].reshape(batch, 128)
    npos = st[:, 0:1]
    loc_abs = st[:, 1:2]
    cep = st[:, 2:3]

    k = jnp.minimum(npos.astype(jnp.int32) * _NEG_POS_RATIO,
                    jnp.int32(num_priors))

    lo = jnp.zeros((batch, 1), jnp.int32)
    hi = jnp.full((batch, 1), 0x7F800000, jnp.int32)
    for _ in range(31):
        mid = lo + ((hi - lo) >> 1)
        t = jax.lax.bitcast_convert_type(mid, jnp.float32)
        cnt = jnp.sum((x >= t).astype(jnp.int32), axis=1, keepdims=True)
        ge = cnt >= k
        lo = jnp.where(ge, mid, lo)
        hi = jnp.where(ge, hi, mid)
    v = jax.lax.bitcast_convert_type(lo, jnp.float32)
    gt = x > v
    cnt_gt = jnp.sum(gt.astype(jnp.float32), axis=1, keepdims=True)
    sum_gt = jnp.sum(jnp.where(gt, x, 0.0), axis=1, keepdims=True)
    hard = sum_gt + (k.astype(jnp.float32) - cnt_gt) * v

    npos_sum = jnp.sum(npos)
    conf_loss = (jnp.sum(hard) + jnp.sum(cep)) / npos_sum
    loc_loss = jnp.sum(loc_abs) / (npos_sum * 4.0)
    out_ref[...] = jnp.full((1, 1), conf_loss + _ALPHA * loc_loss,
                            dtype=jnp.float32)


def kernel(predicted_locs, predicted_scores, boxes, labels, priors_cxcy):
    batch, num_priors, _ = predicted_locs.shape
    num_classes = predicted_scores.shape[2]
    nobj = boxes.shape[1]

    locs_t = jnp.transpose(predicted_locs, (0, 2, 1))
    scores_t = jnp.transpose(predicted_scores, (0, 2, 1))
    priors_t = jnp.transpose(priors_cxcy, (1, 0))
    labels3 = labels.astype(jnp.int32)[..., None]

    conf_neg, stats = pl.pallas_call(
        _phase1_body,
        grid=(batch,),
        in_specs=[
            pl.BlockSpec((1, nobj, 4), lambda i: (i, 0, 0)),
            pl.BlockSpec((1, nobj, 1), lambda i: (i, 0, 0)),
            pl.BlockSpec((4, num_priors), lambda i: (0, 0)),
            pl.BlockSpec((1, 4, num_priors), lambda i: (i, 0, 0)),
            pl.BlockSpec((1, num_classes, num_priors), lambda i: (i, 0, 0)),
        ],
        out_specs=[
            pl.BlockSpec((1, 1, num_priors), lambda i: (i, 0, 0)),
            pl.BlockSpec((1, 1, 128), lambda i: (i, 0, 0)),
        ],
        out_shape=[
            jax.ShapeDtypeStruct((batch, 1, num_priors), jnp.float32),
            jax.ShapeDtypeStruct((batch, 1, 128), jnp.float32),
        ],
    )(boxes, labels3, priors_t, locs_t, scores_t)

    out = pl.pallas_call(
        _phase2_body,
        out_shape=jax.ShapeDtypeStruct((1, 1), jnp.float32),
    )(conf_neg, stats)
    return out[0, 0]
